# Initial kernel scaffold; baseline (speedup 1.0000x reference)
import jax
import jax.numpy as jnp
from jax.experimental import pallas as pl


def kernel(X, mask, W_n, b_n, g_n, be_n, W_e, b_e, g_e, be_e):
    raise NotImplementedError("write your pallas kernel here")


# speedup vs baseline: 1.8312x; 1.8312x over previous
import functools
import math

import jax
import jax.numpy as jnp
from jax.experimental import pallas as pl
from jax.experimental.pallas import tpu as pltpu

TOP_K = 30
NUM_RBF = 16
NUM_POS = 16
BLK = 128


def _normalize(x, eps=1e-12):
    n2 = jnp.sum(x * x, axis=-1, keepdims=True)
    n = jnp.sqrt(jnp.maximum(n2, eps * eps))
    return x / n


def _sign(x):
    return jnp.where(x > 0.0, 1.0, jnp.where(x < 0.0, -1.0, 0.0))


def _ln(x, g, b, eps=1e-5):
    mu = jnp.mean(x, axis=-1, keepdims=True)
    var = jnp.mean((x - mu) ** 2, axis=-1, keepdims=True)
    return (x - mu) / jnp.sqrt(var + eps) * g + b


def _fused_kernel(gr_ref, gall_ref, xat_ref, v_ref,
                  wn_ref, bn_ref, gn_ref, ben_ref,
                  we_ref, be_ref, ge_ref, bee_ref,
                  hv_ref, he_ref, eidx_ref):
    i = pl.program_id(1)
    n = xat_ref.shape[2]

    grow = gr_ref[0]
    xat = xat_ref[0]
    gall = gall_ref[0]

    xr = grow[:, 0:3]
    om = grow[:, 3:12]

    d2 = jnp.zeros((BLK, n), dtype=jnp.float32)
    for c in range(3):
        diff = xr[:, c:c + 1] - xat[c:c + 1, :]
        d2 = d2 + diff * diff
    dmat = jnp.sqrt(d2 + 1e-6)

    iota_n = jax.lax.broadcasted_iota(jnp.int32, (BLK, n), 1)

    dmu = (jax.lax.broadcasted_iota(jnp.int32, (1, NUM_RBF), 1)
           .astype(jnp.float32) * (20.0 / (NUM_RBF - 1)))
    freq = jnp.exp(jax.lax.broadcasted_iota(jnp.int32, (1, NUM_POS // 2), 1)
                   .astype(jnp.float32)
                   * (2.0 * (-math.log(10000.0) / NUM_POS)))
    row_i = (jax.lax.broadcasted_iota(jnp.int32, (BLK, 1), 0)
             .astype(jnp.float32) + jnp.float32(i * BLK))

    we = we_ref[...]
    be = be_ref[...]
    ge = ge_ref[...]
    bee = bee_ref[...]

    dw = dmat
    for k in range(TOP_K):
        m = jnp.min(dw, axis=1, keepdims=True)
        eq = dw == m
        idx = jnp.min(jnp.where(eq, iota_n, n), axis=1, keepdims=True)
        sel = iota_n == idx
        dw = jnp.where(sel, jnp.float32(jnp.inf), dw)

        eidx_ref[0, :, k:k + 1] = idx

        onehot = sel.astype(jnp.float32)
        g = jax.lax.dot_general(onehot, gall, (((1,), (0,)), ((), ())),
                                precision=jax.lax.Precision.HIGHEST,
                                preferred_element_type=jnp.float32)
        xn = g[:, 0:3]
        on = g[:, 3:12]

        omb = om.astype(jnp.bfloat16).astype(jnp.float32)
        onb = on.astype(jnp.bfloat16).astype(jnp.float32)

        dxn = (xn - xr).astype(jnp.bfloat16).astype(jnp.float32)
        du = []
        for a in range(3):
            acc = omb[:, 3 * a:3 * a + 1] * dxn[:, 0:1]
            acc = acc + omb[:, 3 * a + 1:3 * a + 2] * dxn[:, 1:2]
            acc = acc + omb[:, 3 * a + 2:3 * a + 3] * dxn[:, 2:3]
            du.append(acc)
        du = _normalize(jnp.concatenate(du, axis=1))

        rm = [[None] * 3 for _ in range(3)]
        for a in range(3):
            for b in range(3):
                acc = omb[:, a:a + 1] * onb[:, b:b + 1]
                acc = acc + omb[:, 3 + a:4 + a] * onb[:, 3 + b:4 + b]
                acc = acc + omb[:, 6 + a:7 + a] * onb[:, 6 + b:7 + b]
                rm[a][b] = acc

        rxx, ryy, rzz = rm[0][0], rm[1][1], rm[2][2]
        magx = 0.5 * jnp.sqrt(jnp.abs(1.0 + rxx - ryy - rzz))
        magy = 0.5 * jnp.sqrt(jnp.abs(1.0 - rxx + ryy - rzz))
        magz = 0.5 * jnp.sqrt(jnp.abs(1.0 - rxx - ryy + rzz))
        qx = _sign(rm[2][1] - rm[1][2]) * magx
        qy = _sign(rm[0][2] - rm[2][0]) * magy
        qz = _sign(rm[1][0] - rm[0][1]) * magz
        qw = jnp.sqrt(jax.nn.relu(1.0 + rxx + ryy + rzz)) / 2.0
        q = _normalize(jnp.concatenate([qx, qy, qz, qw], axis=1))

        rbf = jnp.exp(-(((m - dmu) / (20.0 / NUM_RBF)) ** 2))

        d_off = idx.astype(jnp.float32) - row_i
        ang = d_off * freq
        e_feat = jnp.concatenate(
            [jnp.cos(ang), jnp.sin(ang), rbf, du, q], axis=1)

        he = jax.lax.dot_general(e_feat, we, (((1,), (0,)), ((), ())),
                                 preferred_element_type=jnp.float32) + be
        he_ref[0, :, k, :] = _ln(he, ge, bee)

    hv = jax.lax.dot_general(v_ref[0], wn_ref[...], (((1,), (0,)), ((), ())),
                             preferred_element_type=jnp.float32) + bn_ref[...]
    hv_ref[0] = _ln(hv, gn_ref[...], ben_ref[...])


@jax.jit
def kernel(X, mask, W_n, b_n, g_n, be_n, W_e, b_e, g_e, be_e):
    B, N = X.shape[0], X.shape[1]
    K = TOP_K
    X_ca = X[:, :, 1, :]

    dX = X_ca[:, 1:, :] - X_ca[:, :-1, :]
    U = _normalize(dX)
    u_2, u_1 = U[:, :-2, :], U[:, 1:-1, :]
    n_2 = _normalize(jnp.cross(u_2, u_1))
    o_1 = _normalize(u_2 - u_1)
    O = jnp.stack([o_1, n_2, jnp.cross(o_1, n_2)], 2).reshape(B, N - 3, 9)
    O = jnp.pad(O, ((0, 0), (1, 2), (0, 0)))

    Xb = X[:, :, :3, :].reshape(B, 3 * N, 3)
    dXb = Xb[:, 1:, :] - Xb[:, :-1, :]
    Ub = _normalize(dXb)
    ub_2, ub_1, ub_0 = Ub[:, :-2, :], Ub[:, 1:-1, :], Ub[:, 2:, :]
    nb_2 = _normalize(jnp.cross(ub_2, ub_1))
    nb_1 = _normalize(jnp.cross(ub_1, ub_0))
    eps2 = 1e-7
    cosDb = jnp.clip(jnp.sum(nb_2 * nb_1, -1), -1 + eps2, 1 - eps2)
    Db = jnp.sign(jnp.sum(ub_2 * nb_1, -1)) * jnp.arccos(cosDb)
    Db = jnp.pad(Db, ((0, 0), (1, 2))).reshape(B, N, 3)
    V = jnp.concatenate([jnp.cos(Db), jnp.sin(Db)], 2)

    gsrc = jnp.concatenate([X_ca, O], axis=2)
    xat = jnp.swapaxes(X_ca, 1, 2)

    grid = (B, N // BLK)
    out = pl.pallas_call(
        _fused_kernel,
        grid=grid,
        in_specs=[
            pl.BlockSpec((1, BLK, 12), lambda b, i: (b, i, 0)),
            pl.BlockSpec((1, N, 12), lambda b, i: (b, 0, 0)),
            pl.BlockSpec((1, 3, N), lambda b, i: (b, 0, 0)),
            pl.BlockSpec((1, BLK, 6), lambda b, i: (b, i, 0)),
            pl.BlockSpec((6, 128), lambda b, i: (0, 0)),
            pl.BlockSpec((1, 128), lambda b, i: (0, 0)),
            pl.BlockSpec((1, 128), lambda b, i: (0, 0)),
            pl.BlockSpec((1, 128), lambda b, i: (0, 0)),
            pl.BlockSpec((39, 128), lambda b, i: (0, 0)),
            pl.BlockSpec((1, 128), lambda b, i: (0, 0)),
            pl.BlockSpec((1, 128), lambda b, i: (0, 0)),
            pl.BlockSpec((1, 128), lambda b, i: (0, 0)),
        ],
        out_specs=[
            pl.BlockSpec((1, BLK, 128), lambda b, i: (b, i, 0)),
            pl.BlockSpec((1, BLK, K, 128), lambda b, i: (b, i, 0, 0)),
            pl.BlockSpec((1, BLK, K), lambda b, i: (b, i, 0)),
        ],
        out_shape=[
            jax.ShapeDtypeStruct((B, N, 128), jnp.float32),
            jax.ShapeDtypeStruct((B, N, K, 128), jnp.float32),
            jax.ShapeDtypeStruct((B, N, K), jnp.int32),
        ],
        compiler_params=pltpu.CompilerParams(
            dimension_semantics=("parallel", "parallel")),
    )(gsrc, gsrc, xat, V,
      W_n, b_n.reshape(1, 128), g_n.reshape(1, 128), be_n.reshape(1, 128),
      W_e, b_e.reshape(1, 128), g_e.reshape(1, 128), be_e.reshape(1, 128))

    h_V, h_E, E_idx = out
    return h_V, h_E, E_idx


# speedup vs baseline: 1.9131x; 1.0447x over previous
import functools
import math

import jax
import jax.numpy as jnp
from jax.experimental import pallas as pl
from jax.experimental.pallas import tpu as pltpu

TOP_K = 30
NUM_RBF = 16
NUM_POS = 16
BLK = 256


def _normalize(x, eps=1e-12):
    n2 = jnp.sum(x * x, axis=-1, keepdims=True)
    n = jnp.sqrt(jnp.maximum(n2, eps * eps))
    return x / n


def _sign(x):
    return jnp.where(x > 0.0, 1.0, jnp.where(x < 0.0, -1.0, 0.0))


def _ln(x, g, b, eps=1e-5):
    mu = jnp.mean(x, axis=-1, keepdims=True)
    var = jnp.mean((x - mu) ** 2, axis=-1, keepdims=True)
    return (x - mu) / jnp.sqrt(var + eps) * g + b


def _fused_kernel(gr_ref, gall_ref, xat_ref, v_ref,
                  wn_ref, bn_ref, gn_ref, ben_ref,
                  we_ref, be_ref, ge_ref, bee_ref,
                  hv_ref, he_ref, eidx_ref):
    i = pl.program_id(1)
    n = xat_ref.shape[2]

    grow = gr_ref[0]
    xat = xat_ref[0]
    gall = gall_ref[0]

    xr = grow[:, 0:3]
    om = grow[:, 3:12]

    d2 = jnp.zeros((BLK, n), dtype=jnp.float32)
    for c in range(3):
        diff = xr[:, c:c + 1] - xat[c:c + 1, :]
        d2 = d2 + diff * diff
    dmat = jnp.sqrt(d2 + 1e-6)

    iota_n = jax.lax.broadcasted_iota(jnp.int32, (BLK, n), 1)

    dmu = (jax.lax.broadcasted_iota(jnp.int32, (1, NUM_RBF), 1)
           .astype(jnp.float32) * (20.0 / (NUM_RBF - 1)))
    freq = jnp.exp(jax.lax.broadcasted_iota(jnp.int32, (1, NUM_POS // 2), 1)
                   .astype(jnp.float32)
                   * (2.0 * (-math.log(10000.0) / NUM_POS)))
    row_i = (jax.lax.broadcasted_iota(jnp.int32, (BLK, 1), 0)
             .astype(jnp.float32) + jnp.float32(i * BLK))

    we = we_ref[...]
    be = be_ref[...]
    ge = ge_ref[...]
    bee = bee_ref[...]

    dw = dmat
    for k in range(TOP_K):
        m = jnp.min(dw, axis=1, keepdims=True)
        eq = dw == m
        idx = jnp.min(jnp.where(eq, iota_n, n), axis=1, keepdims=True)
        sel = iota_n == idx
        dw = jnp.where(sel, jnp.float32(jnp.inf), dw)

        eidx_ref[0, :, k:k + 1] = idx

        onehot = sel.astype(jnp.float32)
        g = jax.lax.dot_general(onehot, gall, (((1,), (0,)), ((), ())),
                                precision=jax.lax.Precision.HIGHEST,
                                preferred_element_type=jnp.float32)
        xn = g[:, 0:3]
        on = g[:, 3:12]

        omb = om.astype(jnp.bfloat16).astype(jnp.float32)
        onb = on.astype(jnp.bfloat16).astype(jnp.float32)

        dxn = (xn - xr).astype(jnp.bfloat16).astype(jnp.float32)
        du = []
        for a in range(3):
            acc = omb[:, 3 * a:3 * a + 1] * dxn[:, 0:1]
            acc = acc + omb[:, 3 * a + 1:3 * a + 2] * dxn[:, 1:2]
            acc = acc + omb[:, 3 * a + 2:3 * a + 3] * dxn[:, 2:3]
            du.append(acc)
        du = _normalize(jnp.concatenate(du, axis=1))

        rm = [[None] * 3 for _ in range(3)]
        for a in range(3):
            for b in range(3):
                acc = omb[:, a:a + 1] * onb[:, b:b + 1]
                acc = acc + omb[:, 3 + a:4 + a] * onb[:, 3 + b:4 + b]
                acc = acc + omb[:, 6 + a:7 + a] * onb[:, 6 + b:7 + b]
                rm[a][b] = acc

        rxx, ryy, rzz = rm[0][0], rm[1][1], rm[2][2]
        magx = 0.5 * jnp.sqrt(jnp.abs(1.0 + rxx - ryy - rzz))
        magy = 0.5 * jnp.sqrt(jnp.abs(1.0 - rxx + ryy - rzz))
        magz = 0.5 * jnp.sqrt(jnp.abs(1.0 - rxx - ryy + rzz))
        qx = _sign(rm[2][1] - rm[1][2]) * magx
        qy = _sign(rm[0][2] - rm[2][0]) * magy
        qz = _sign(rm[1][0] - rm[0][1]) * magz
        qw = jnp.sqrt(jax.nn.relu(1.0 + rxx + ryy + rzz)) / 2.0
        q = _normalize(jnp.concatenate([qx, qy, qz, qw], axis=1))

        rbf = jnp.exp(-(((m - dmu) / (20.0 / NUM_RBF)) ** 2))

        d_off = idx.astype(jnp.float32) - row_i
        ang = d_off * freq
        e_feat = jnp.concatenate(
            [jnp.cos(ang), jnp.sin(ang), rbf, du, q], axis=1)

        he = jax.lax.dot_general(e_feat, we, (((1,), (0,)), ((), ())),
                                 preferred_element_type=jnp.float32) + be
        he_ref[0, :, k, :] = _ln(he, ge, bee)

    hv = jax.lax.dot_general(v_ref[0], wn_ref[...], (((1,), (0,)), ((), ())),
                             preferred_element_type=jnp.float32) + bn_ref[...]
    hv_ref[0] = _ln(hv, gn_ref[...], ben_ref[...])


@jax.jit
def kernel(X, mask, W_n, b_n, g_n, be_n, W_e, b_e, g_e, be_e):
    B, N = X.shape[0], X.shape[1]
    K = TOP_K
    X_ca = X[:, :, 1, :]

    dX = X_ca[:, 1:, :] - X_ca[:, :-1, :]
    U = _normalize(dX)
    u_2, u_1 = U[:, :-2, :], U[:, 1:-1, :]
    n_2 = _normalize(jnp.cross(u_2, u_1))
    o_1 = _normalize(u_2 - u_1)
    O = jnp.stack([o_1, n_2, jnp.cross(o_1, n_2)], 2).reshape(B, N - 3, 9)
    O = jnp.pad(O, ((0, 0), (1, 2), (0, 0)))

    Xb = X[:, :, :3, :].reshape(B, 3 * N, 3)
    dXb = Xb[:, 1:, :] - Xb[:, :-1, :]
    Ub = _normalize(dXb)
    ub_2, ub_1, ub_0 = Ub[:, :-2, :], Ub[:, 1:-1, :], Ub[:, 2:, :]
    nb_2 = _normalize(jnp.cross(ub_2, ub_1))
    nb_1 = _normalize(jnp.cross(ub_1, ub_0))
    eps2 = 1e-7
    cosDb = jnp.clip(jnp.sum(nb_2 * nb_1, -1), -1 + eps2, 1 - eps2)
    Db = jnp.sign(jnp.sum(ub_2 * nb_1, -1)) * jnp.arccos(cosDb)
    Db = jnp.pad(Db, ((0, 0), (1, 2))).reshape(B, N, 3)
    V = jnp.concatenate([jnp.cos(Db), jnp.sin(Db)], 2)

    gsrc = jnp.concatenate([X_ca, O], axis=2)
    xat = jnp.swapaxes(X_ca, 1, 2)

    grid = (B, N // BLK)
    out = pl.pallas_call(
        _fused_kernel,
        grid=grid,
        in_specs=[
            pl.BlockSpec((1, BLK, 12), lambda b, i: (b, i, 0)),
            pl.BlockSpec((1, N, 12), lambda b, i: (b, 0, 0)),
            pl.BlockSpec((1, 3, N), lambda b, i: (b, 0, 0)),
            pl.BlockSpec((1, BLK, 6), lambda b, i: (b, i, 0)),
            pl.BlockSpec((6, 128), lambda b, i: (0, 0)),
            pl.BlockSpec((1, 128), lambda b, i: (0, 0)),
            pl.BlockSpec((1, 128), lambda b, i: (0, 0)),
            pl.BlockSpec((1, 128), lambda b, i: (0, 0)),
            pl.BlockSpec((39, 128), lambda b, i: (0, 0)),
            pl.BlockSpec((1, 128), lambda b, i: (0, 0)),
            pl.BlockSpec((1, 128), lambda b, i: (0, 0)),
            pl.BlockSpec((1, 128), lambda b, i: (0, 0)),
        ],
        out_specs=[
            pl.BlockSpec((1, BLK, 128), lambda b, i: (b, i, 0)),
            pl.BlockSpec((1, BLK, K, 128), lambda b, i: (b, i, 0, 0)),
            pl.BlockSpec((1, BLK, K), lambda b, i: (b, i, 0)),
        ],
        out_shape=[
            jax.ShapeDtypeStruct((B, N, 128), jnp.float32),
            jax.ShapeDtypeStruct((B, N, K, 128), jnp.float32),
            jax.ShapeDtypeStruct((B, N, K), jnp.int32),
        ],
        compiler_params=pltpu.CompilerParams(
            dimension_semantics=("parallel", "parallel")),
    )(gsrc, gsrc, xat, V,
      W_n, b_n.reshape(1, 128), g_n.reshape(1, 128), be_n.reshape(1, 128),
      W_e, b_e.reshape(1, 128), g_e.reshape(1, 128), be_e.reshape(1, 128))

    h_V, h_E, E_idx = out
    return h_V, h_E, E_idx


# speedup vs baseline: 1.9456x; 1.0170x over previous
import functools
import math

import jax
import jax.numpy as jnp
from jax.experimental import pallas as pl
from jax.experimental.pallas import tpu as pltpu

TOP_K = 30
NUM_RBF = 16
NUM_POS = 16
BLK = 512


def _normalize(x, eps=1e-12):
    n2 = jnp.sum(x * x, axis=-1, keepdims=True)
    n = jnp.sqrt(jnp.maximum(n2, eps * eps))
    return x / n


def _sign(x):
    return jnp.where(x > 0.0, 1.0, jnp.where(x < 0.0, -1.0, 0.0))


def _ln(x, g, b, eps=1e-5):
    mu = jnp.mean(x, axis=-1, keepdims=True)
    var = jnp.mean((x - mu) ** 2, axis=-1, keepdims=True)
    return (x - mu) / jnp.sqrt(var + eps) * g + b


def _fused_kernel(gr_ref, gall_ref, xat_ref, v_ref,
                  wn_ref, bn_ref, gn_ref, ben_ref,
                  we_ref, be_ref, ge_ref, bee_ref,
                  hv_ref, he_ref, eidx_ref):
    i = pl.program_id(1)
    n = xat_ref.shape[2]

    grow = gr_ref[0]
    xat = xat_ref[0]
    gall = gall_ref[0]

    xr = grow[:, 0:3]
    om = grow[:, 3:12]

    d2 = jnp.zeros((BLK, n), dtype=jnp.float32)
    for c in range(3):
        diff = xr[:, c:c + 1] - xat[c:c + 1, :]
        d2 = d2 + diff * diff
    dmat = jnp.sqrt(d2 + 1e-6)

    iota_n = jax.lax.broadcasted_iota(jnp.int32, (BLK, n), 1)

    dmu = (jax.lax.broadcasted_iota(jnp.int32, (1, NUM_RBF), 1)
           .astype(jnp.float32) * (20.0 / (NUM_RBF - 1)))
    freq = jnp.exp(jax.lax.broadcasted_iota(jnp.int32, (1, NUM_POS // 2), 1)
                   .astype(jnp.float32)
                   * (2.0 * (-math.log(10000.0) / NUM_POS)))
    row_i = (jax.lax.broadcasted_iota(jnp.int32, (BLK, 1), 0)
             .astype(jnp.float32) + jnp.float32(i * BLK))

    we = we_ref[...]
    be = be_ref[...]
    ge = ge_ref[...]
    bee = bee_ref[...]

    dw = dmat
    for k in range(TOP_K):
        m = jnp.min(dw, axis=1, keepdims=True)
        eq = dw == m
        idx = jnp.min(jnp.where(eq, iota_n, n), axis=1, keepdims=True)
        sel = iota_n == idx
        dw = jnp.where(sel, jnp.float32(jnp.inf), dw)

        eidx_ref[0, :, k:k + 1] = idx

        onehot = sel.astype(jnp.float32)
        g = jax.lax.dot_general(onehot, gall, (((1,), (0,)), ((), ())),
                                precision=jax.lax.Precision.HIGHEST,
                                preferred_element_type=jnp.float32)
        xn = g[:, 0:3]
        on = g[:, 3:12]

        omb = om.astype(jnp.bfloat16).astype(jnp.float32)
        onb = on.astype(jnp.bfloat16).astype(jnp.float32)

        dxn = (xn - xr).astype(jnp.bfloat16).astype(jnp.float32)
        du = []
        for a in range(3):
            acc = omb[:, 3 * a:3 * a + 1] * dxn[:, 0:1]
            acc = acc + omb[:, 3 * a + 1:3 * a + 2] * dxn[:, 1:2]
            acc = acc + omb[:, 3 * a + 2:3 * a + 3] * dxn[:, 2:3]
            du.append(acc)
        du = _normalize(jnp.concatenate(du, axis=1))

        rm = [[None] * 3 for _ in range(3)]
        for a in range(3):
            for b in range(3):
                acc = omb[:, a:a + 1] * onb[:, b:b + 1]
                acc = acc + omb[:, 3 + a:4 + a] * onb[:, 3 + b:4 + b]
                acc = acc + omb[:, 6 + a:7 + a] * onb[:, 6 + b:7 + b]
                rm[a][b] = acc

        rxx, ryy, rzz = rm[0][0], rm[1][1], rm[2][2]
        magx = 0.5 * jnp.sqrt(jnp.abs(1.0 + rxx - ryy - rzz))
        magy = 0.5 * jnp.sqrt(jnp.abs(1.0 - rxx + ryy - rzz))
        magz = 0.5 * jnp.sqrt(jnp.abs(1.0 - rxx - ryy + rzz))
        qx = _sign(rm[2][1] - rm[1][2]) * magx
        qy = _sign(rm[0][2] - rm[2][0]) * magy
        qz = _sign(rm[1][0] - rm[0][1]) * magz
        qw = jnp.sqrt(jax.nn.relu(1.0 + rxx + ryy + rzz)) / 2.0
        q = _normalize(jnp.concatenate([qx, qy, qz, qw], axis=1))

        rbf = jnp.exp(-(((m - dmu) / (20.0 / NUM_RBF)) ** 2))

        d_off = idx.astype(jnp.float32) - row_i
        ang = d_off * freq
        e_feat = jnp.concatenate(
            [jnp.cos(ang), jnp.sin(ang), rbf, du, q], axis=1)

        he = jax.lax.dot_general(e_feat, we, (((1,), (0,)), ((), ())),
                                 preferred_element_type=jnp.float32) + be
        he_ref[0, :, k, :] = _ln(he, ge, bee)

    hv = jax.lax.dot_general(v_ref[0], wn_ref[...], (((1,), (0,)), ((), ())),
                             preferred_element_type=jnp.float32) + bn_ref[...]
    hv_ref[0] = _ln(hv, gn_ref[...], ben_ref[...])


@jax.jit
def kernel(X, mask, W_n, b_n, g_n, be_n, W_e, b_e, g_e, be_e):
    B, N = X.shape[0], X.shape[1]
    K = TOP_K
    X_ca = X[:, :, 1, :]

    dX = X_ca[:, 1:, :] - X_ca[:, :-1, :]
    U = _normalize(dX)
    u_2, u_1 = U[:, :-2, :], U[:, 1:-1, :]
    n_2 = _normalize(jnp.cross(u_2, u_1))
    o_1 = _normalize(u_2 - u_1)
    O = jnp.stack([o_1, n_2, jnp.cross(o_1, n_2)], 2).reshape(B, N - 3, 9)
    O = jnp.pad(O, ((0, 0), (1, 2), (0, 0)))

    Xb = X[:, :, :3, :].reshape(B, 3 * N, 3)
    dXb = Xb[:, 1:, :] - Xb[:, :-1, :]
    Ub = _normalize(dXb)
    ub_2, ub_1, ub_0 = Ub[:, :-2, :], Ub[:, 1:-1, :], Ub[:, 2:, :]
    nb_2 = _normalize(jnp.cross(ub_2, ub_1))
    nb_1 = _normalize(jnp.cross(ub_1, ub_0))
    eps2 = 1e-7
    cosDb = jnp.clip(jnp.sum(nb_2 * nb_1, -1), -1 + eps2, 1 - eps2)
    Db = jnp.sign(jnp.sum(ub_2 * nb_1, -1)) * jnp.arccos(cosDb)
    Db = jnp.pad(Db, ((0, 0), (1, 2))).reshape(B, N, 3)
    V = jnp.concatenate([jnp.cos(Db), jnp.sin(Db)], 2)

    gsrc = jnp.concatenate([X_ca, O], axis=2)
    xat = jnp.swapaxes(X_ca, 1, 2)

    grid = (B, N // BLK)
    out = pl.pallas_call(
        _fused_kernel,
        grid=grid,
        in_specs=[
            pl.BlockSpec((1, BLK, 12), lambda b, i: (b, i, 0)),
            pl.BlockSpec((1, N, 12), lambda b, i: (b, 0, 0)),
            pl.BlockSpec((1, 3, N), lambda b, i: (b, 0, 0)),
            pl.BlockSpec((1, BLK, 6), lambda b, i: (b, i, 0)),
            pl.BlockSpec((6, 128), lambda b, i: (0, 0)),
            pl.BlockSpec((1, 128), lambda b, i: (0, 0)),
            pl.BlockSpec((1, 128), lambda b, i: (0, 0)),
            pl.BlockSpec((1, 128), lambda b, i: (0, 0)),
            pl.BlockSpec((39, 128), lambda b, i: (0, 0)),
            pl.BlockSpec((1, 128), lambda b, i: (0, 0)),
            pl.BlockSpec((1, 128), lambda b, i: (0, 0)),
            pl.BlockSpec((1, 128), lambda b, i: (0, 0)),
        ],
        out_specs=[
            pl.BlockSpec((1, BLK, 128), lambda b, i: (b, i, 0)),
            pl.BlockSpec((1, BLK, K, 128), lambda b, i: (b, i, 0, 0)),
            pl.BlockSpec((1, BLK, K), lambda b, i: (b, i, 0)),
        ],
        out_shape=[
            jax.ShapeDtypeStruct((B, N, 128), jnp.float32),
            jax.ShapeDtypeStruct((B, N, K, 128), jnp.float32),
            jax.ShapeDtypeStruct((B, N, K), jnp.int32),
        ],
        compiler_params=pltpu.CompilerParams(
            dimension_semantics=("parallel", "parallel")),
    )(gsrc, gsrc, xat, V,
      W_n, b_n.reshape(1, 128), g_n.reshape(1, 128), be_n.reshape(1, 128),
      W_e, b_e.reshape(1, 128), g_e.reshape(1, 128), be_e.reshape(1, 128))

    h_V, h_E, E_idx = out
    return h_V, h_E, E_idx
